# baseline (speedup 1.0000x reference)
import functools

import jax
import jax.numpy as jnp
from jax import lax
from jax.experimental import pallas as pl
from jax.experimental.pallas import tpu as pltpu
from jax.experimental.pallas import tpu_sc as plsc

NUM_CORES = 2
NUM_SUBCORES = 16
NW = NUM_CORES * NUM_SUBCORES
BATCH = 16384
B_PER_W = BATCH // NW
N_CHUNKS = 4
CHUNK = B_PER_W // N_CHUNKS
EMBED = 32
LANES = 16
ROW_UNROLL = 4


def _sc_body(user_hbm, item_hbm, user_table, item_table, out_hbm,
             uidx, iidx, urows, irows, out_v, sem):
    c = lax.axis_index("c")
    s = lax.axis_index("s")
    wid = s * NUM_CORES + c
    base = wid * B_PER_W

    pltpu.sync_copy(user_hbm.at[wid], uidx)
    pltpu.sync_copy(item_hbm.at[wid], iidx)

    copies = []
    for j in range(N_CHUNKS):
        copies.append(pltpu.async_copy(
            user_table.at[uidx.at[j]], urows.at[pl.ds(j * CHUNK, CHUNK)], sem))
        copies.append(pltpu.async_copy(
            item_table.at[iidx.at[j]], irows.at[pl.ds(j * CHUNK, CHUNK)], sem))
    for cp in copies:
        cp.wait()

    last_lane = lax.iota(jnp.int32, LANES) == (LANES - 1)

    def body(i, carry):
        for k in range(ROW_UNROLL):
            b = i * ROW_UNROLL + k
            u0 = urows[b, pl.ds(0, LANES)]
            u1 = urows[b, pl.ds(LANES, LANES)]
            v0 = irows[b, pl.ds(0, LANES)]
            v1 = irows[b, pl.ds(LANES, LANES)]
            cs = plsc.cumsum(u0 * v0 + u1 * v1)
            plsc.store_scatter(out_v, [jnp.full((LANES,), b, jnp.int32)],
                               cs, mask=last_lane)
        return carry

    lax.fori_loop(0, B_PER_W // ROW_UNROLL, body, 0)

    pltpu.sync_copy(out_v, out_hbm.at[pl.ds(base, B_PER_W)])


@functools.partial(
    pl.kernel,
    out_type=jax.ShapeDtypeStruct((BATCH,), jnp.float32),
    mesh=plsc.VectorSubcoreMesh(core_axis_name="c", subcore_axis_name="s"),
    scratch_types=[
        pltpu.VMEM((N_CHUNKS, CHUNK), jnp.int32),
        pltpu.VMEM((N_CHUNKS, CHUNK), jnp.int32),
        pltpu.VMEM((B_PER_W, EMBED), jnp.float32),
        pltpu.VMEM((B_PER_W, EMBED), jnp.float32),
        pltpu.VMEM((B_PER_W,), jnp.float32),
        pltpu.SemaphoreType.DMA,
    ],
    compiler_params=pltpu.CompilerParams(needs_layout_passes=False,
                                         use_tc_tiling_on_sc=False),
)
def _sc_kernel(user_hbm, item_hbm, user_table, item_table, out_hbm,
               uidx, iidx, urows, irows, out_v, sem):
    _sc_body(user_hbm, item_hbm, user_table, item_table, out_hbm,
             uidx, iidx, urows, irows, out_v, sem)


def kernel(user, item, user_table, item_table):
    user_r = user.astype(jnp.int32).reshape(NW, N_CHUNKS, CHUNK)
    item_r = item.astype(jnp.int32).reshape(NW, N_CHUNKS, CHUNK)
    return _sc_kernel(user_r, item_r, user_table, item_table)


# speedup vs baseline: 4.2641x; 4.2641x over previous
import functools

import jax
import jax.numpy as jnp
from jax import lax
from jax.experimental import pallas as pl
from jax.experimental.pallas import tpu as pltpu
from jax.experimental.pallas import tpu_sc as plsc

NUM_CORES = 2
NUM_SUBCORES = 16
NW = NUM_CORES * NUM_SUBCORES
BATCH = 16384
B_PER_W = BATCH // NW
EMBED = 32
LANES = 16
DEPTH = 4
GROUPS = B_PER_W // LANES


def _off(c):
    return pl.multiple_of((c >> 7) * 128, 128)


def _issue(tab, c, win, slot, sem):
    pltpu.async_copy(tab.at[:, pl.ds(_off(c), 128)], win.at[slot], sem)


def _retire(tab, c, win, slot, sem):
    pltpu.make_async_copy(tab.at[:, pl.ds(_off(c), 128)],
                          win.at[slot], sem).wait()


def _extract(win, slot, c, res, kp):
    rows = lax.iota(jnp.int32, LANES)
    cols = jnp.full((LANES,), c & 127, jnp.int32)
    kcols = jnp.full((LANES,), kp, jnp.int32)
    lo = plsc.load_gather(win.at[slot], [rows, cols])
    hi = plsc.load_gather(win.at[slot], [rows + LANES, cols])
    plsc.store_scatter(res, [rows, kcols], lo)
    plsc.store_scatter(res, [rows + LANES, kcols], hi)


def _sc_body(user_hbm, item_hbm, ut_t, it_t, out_hbm,
             uidx_v, iidx_v, uwin, iwin, ures, ires, out_v, sem):
    core = lax.axis_index("c")
    sub = lax.axis_index("s")
    wid = sub * NUM_CORES + core
    base = wid * B_PER_W

    pltpu.sync_copy(user_hbm.at[pl.ds(base, B_PER_W)], uidx_v)
    pltpu.sync_copy(item_hbm.at[pl.ds(base, B_PER_W)], iidx_v)

    def step(cu, cv, cu_prev, cv_prev, g, j):
        if j >= DEPTH:
            cpu, cpv = cu[j - DEPTH], cv[j - DEPTH]
        else:
            cpu, cpv = cu_prev[LANES - DEPTH + j], cv_prev[LANES - DEPTH + j]
        slot = (j - DEPTH) % DEPTH
        kp = g * LANES + j - DEPTH
        _retire(ut_t, cpu, uwin, slot, sem)
        _retire(it_t, cpv, iwin, slot, sem)
        _extract(uwin, slot, cpu, ures, kp)
        _extract(iwin, slot, cpv, ires, kp)
        _issue(ut_t, cu[j], uwin, j % DEPTH, sem)
        _issue(it_t, cv[j], iwin, j % DEPTH, sem)

    cu0 = uidx_v[pl.ds(0, LANES)]
    cv0 = iidx_v[pl.ds(0, LANES)]
    for j in range(DEPTH):
        _issue(ut_t, cu0[j], uwin, j, sem)
        _issue(it_t, cv0[j], iwin, j, sem)
    for j in range(DEPTH, LANES):
        step(cu0, cv0, cu0, cv0, 0, j)

    def body(g, carry):
        cu_prev, cv_prev = carry
        cu = uidx_v[pl.ds(g * LANES, LANES)]
        cv = iidx_v[pl.ds(g * LANES, LANES)]
        for j in range(LANES):
            step(cu, cv, cu_prev, cv_prev, g, j)
        return (cu, cv)

    cu_last, cv_last = lax.fori_loop(1, GROUPS, body, (cu0, cv0))

    for j in range(DEPTH):
        cpu = cu_last[LANES - DEPTH + j]
        cpv = cv_last[LANES - DEPTH + j]
        slot = j % DEPTH
        kp = B_PER_W - DEPTH + j
        _retire(ut_t, cpu, uwin, slot, sem)
        _retire(it_t, cpv, iwin, slot, sem)
        _extract(uwin, slot, cpu, ures, kp)
        _extract(iwin, slot, cpv, ires, kp)

    def dot_body(g, carry):
        b0 = g * LANES
        acc = ures[0, pl.ds(b0, LANES)] * ires[0, pl.ds(b0, LANES)]
        for d in range(1, EMBED):
            acc = acc + ures[d, pl.ds(b0, LANES)] * ires[d, pl.ds(b0, LANES)]
        out_v[pl.ds(b0, LANES)] = acc
        return carry

    lax.fori_loop(0, GROUPS, dot_body, 0)

    pltpu.sync_copy(out_v, out_hbm.at[pl.ds(base, B_PER_W)])


@functools.partial(
    pl.kernel,
    out_type=jax.ShapeDtypeStruct((BATCH,), jnp.float32),
    mesh=plsc.VectorSubcoreMesh(core_axis_name="c", subcore_axis_name="s"),
    scratch_types=[
        pltpu.VMEM((B_PER_W,), jnp.int32),
        pltpu.VMEM((B_PER_W,), jnp.int32),
        pltpu.VMEM((DEPTH, EMBED, 128), jnp.float32),
        pltpu.VMEM((DEPTH, EMBED, 128), jnp.float32),
        pltpu.VMEM((EMBED, B_PER_W), jnp.float32),
        pltpu.VMEM((EMBED, B_PER_W), jnp.float32),
        pltpu.VMEM((B_PER_W,), jnp.float32),
        pltpu.SemaphoreType.DMA,
    ],
    compiler_params=pltpu.CompilerParams(needs_layout_passes=False),
)
def _sc_kernel(user_hbm, item_hbm, ut_t, it_t, out_hbm, *scratch):
    _sc_body(user_hbm, item_hbm, ut_t, it_t, out_hbm, *scratch)


def kernel(user, item, user_table, item_table):
    return _sc_kernel(user.astype(jnp.int32), item.astype(jnp.int32),
                      user_table.T, item_table.T)


# speedup vs baseline: 4.5184x; 1.0597x over previous
import functools

import jax
import jax.numpy as jnp
from jax import lax
from jax.experimental import pallas as pl
from jax.experimental.pallas import tpu as pltpu
from jax.experimental.pallas import tpu_sc as plsc

NUM_CORES = 2
NUM_SUBCORES = 16
NW = NUM_CORES * NUM_SUBCORES
BATCH = 16384
B_PER_W = BATCH // NW
EMBED = 32
LANES = 16
DEPTH = 8
GROUPS = B_PER_W // LANES


def _off(c):
    return pl.multiple_of((c >> 7) * 128, 128)


def _issue(tab, c, win, slot, sem):
    pltpu.async_copy(tab.at[:, pl.ds(_off(c), 128)], win.at[slot], sem)


def _retire(tab, c, win, slot, sem):
    pltpu.make_async_copy(tab.at[:, pl.ds(_off(c), 128)],
                          win.at[slot], sem).wait()


def _extract(win, slot, c, res, kp):
    rows = lax.iota(jnp.int32, LANES)
    cols = jnp.full((LANES,), c & 127, jnp.int32)
    kcols = jnp.full((LANES,), kp, jnp.int32)
    lo = plsc.load_gather(win.at[slot], [rows, cols])
    hi = plsc.load_gather(win.at[slot], [rows + LANES, cols])
    plsc.store_scatter(res, [rows, kcols], lo)
    plsc.store_scatter(res, [rows + LANES, kcols], hi)


def _sc_body(user_hbm, item_hbm, ut_t, it_t, out_hbm,
             uidx_v, iidx_v, uwin, iwin, ures, ires, out_v, sem):
    core = lax.axis_index("c")
    sub = lax.axis_index("s")
    wid = sub * NUM_CORES + core
    base = wid * B_PER_W

    pltpu.sync_copy(user_hbm.at[pl.ds(base, B_PER_W)], uidx_v)
    pltpu.sync_copy(item_hbm.at[pl.ds(base, B_PER_W)], iidx_v)

    def step(cu, cv, cu_prev, cv_prev, g, j):
        if j >= DEPTH:
            cpu, cpv = cu[j - DEPTH], cv[j - DEPTH]
        else:
            cpu, cpv = cu_prev[LANES - DEPTH + j], cv_prev[LANES - DEPTH + j]
        slot = (j - DEPTH) % DEPTH
        kp = g * LANES + j - DEPTH
        _retire(ut_t, cpu, uwin, slot, sem)
        _retire(it_t, cpv, iwin, slot, sem)
        _extract(uwin, slot, cpu, ures, kp)
        _extract(iwin, slot, cpv, ires, kp)
        _issue(ut_t, cu[j], uwin, j % DEPTH, sem)
        _issue(it_t, cv[j], iwin, j % DEPTH, sem)

    cu0 = uidx_v[pl.ds(0, LANES)]
    cv0 = iidx_v[pl.ds(0, LANES)]
    for j in range(DEPTH):
        _issue(ut_t, cu0[j], uwin, j, sem)
        _issue(it_t, cv0[j], iwin, j, sem)
    for j in range(DEPTH, LANES):
        step(cu0, cv0, cu0, cv0, 0, j)

    def body(g, carry):
        cu_prev, cv_prev = carry
        cu = uidx_v[pl.ds(g * LANES, LANES)]
        cv = iidx_v[pl.ds(g * LANES, LANES)]
        for j in range(LANES):
            step(cu, cv, cu_prev, cv_prev, g, j)
        return (cu, cv)

    cu_last, cv_last = lax.fori_loop(1, GROUPS, body, (cu0, cv0))

    for j in range(DEPTH):
        cpu = cu_last[LANES - DEPTH + j]
        cpv = cv_last[LANES - DEPTH + j]
        slot = j % DEPTH
        kp = B_PER_W - DEPTH + j
        _retire(ut_t, cpu, uwin, slot, sem)
        _retire(it_t, cpv, iwin, slot, sem)
        _extract(uwin, slot, cpu, ures, kp)
        _extract(iwin, slot, cpv, ires, kp)

    def dot_body(g, carry):
        b0 = g * LANES
        acc = ures[0, pl.ds(b0, LANES)] * ires[0, pl.ds(b0, LANES)]
        for d in range(1, EMBED):
            acc = acc + ures[d, pl.ds(b0, LANES)] * ires[d, pl.ds(b0, LANES)]
        out_v[pl.ds(b0, LANES)] = acc
        return carry

    lax.fori_loop(0, GROUPS, dot_body, 0)

    pltpu.sync_copy(out_v, out_hbm.at[pl.ds(base, B_PER_W)])


@functools.partial(
    pl.kernel,
    out_type=jax.ShapeDtypeStruct((BATCH,), jnp.float32),
    mesh=plsc.VectorSubcoreMesh(core_axis_name="c", subcore_axis_name="s"),
    scratch_types=[
        pltpu.VMEM((B_PER_W,), jnp.int32),
        pltpu.VMEM((B_PER_W,), jnp.int32),
        pltpu.VMEM((DEPTH, EMBED, 128), jnp.float32),
        pltpu.VMEM((DEPTH, EMBED, 128), jnp.float32),
        pltpu.VMEM((EMBED, B_PER_W), jnp.float32),
        pltpu.VMEM((EMBED, B_PER_W), jnp.float32),
        pltpu.VMEM((B_PER_W,), jnp.float32),
        pltpu.SemaphoreType.DMA,
    ],
    compiler_params=pltpu.CompilerParams(needs_layout_passes=False),
)
def _sc_kernel(user_hbm, item_hbm, ut_t, it_t, out_hbm, *scratch):
    _sc_body(user_hbm, item_hbm, ut_t, it_t, out_hbm, *scratch)


def kernel(user, item, user_table, item_table):
    return _sc_kernel(user.astype(jnp.int32), item.astype(jnp.int32),
                      user_table.T, item_table.T)
